# baseline (speedup 1.0000x reference)
import functools

import jax
import jax.numpy as jnp
from jax import lax
from jax.experimental import pallas as pl
from jax.experimental.pallas import tpu as pltpu
from jax.experimental.pallas import tpu_sc as plsc

N = 10000
NPAD = 10240
E = 320000
EPAD = 327680
NW = 32
EPW = EPAD // NW
CHUNK = 2048
NCH = EPW // CHUNK
KCAP2 = 10496
DUMPSLOT = NPAD * NPAD - 1
R2 = 1.0
PT_THLD = 0.9
BT = 512
BS = 512

_mesh = plsc.VectorSubcoreMesh(core_axis_name="c", subcore_axis_name="s")
_sc_params = pltpu.CompilerParams(needs_layout_passes=False)


def _scatter_body(src_hbm, tgt_hbm, pt_hbm,
                  table_hbm, keys_hbm, srcs_hbm, tgts_hbm, cnts_hbm,
                  ptv, srcv, tgtv, keyflat, sflat, tflat, ids2d, cntbuf, sem):
    c = lax.axis_index("c")
    s_ = lax.axis_index("s")
    w = c * 16 + s_
    pltpu.sync_copy(pt_hbm, ptv)
    ebase = w * EPW

    def fill_ids(j, _):
        for k in range(8):
            ids2d[j, pl.ds(k * 16, 16)] = (
                lax.iota(jnp.int32, 16) + (w * KCAP2 + j * 128 + k * 16))
        return 0

    lax.fori_loop(0, KCAP2 // 128, fill_ids, 0)

    def do_chunk(ch, off):
        pltpu.sync_copy(src_hbm.at[pl.ds(ebase + ch * CHUNK, CHUNK)], srcv)
        pltpu.sync_copy(tgt_hbm.at[pl.ds(ebase + ch * CHUNK, CHUNK)], tgtv)

        def vec(i, off):
            sidx = srcv[pl.ds(i * 16, 16)]
            tidx = tgtv[pl.ds(i * 16, 16)]
            ptg = plsc.load_gather(ptv, [sidx])
            m = ptg > PT_THLD
            key = tidx * NPAD + sidx
            plsc.store_compressed(keyflat.at[pl.ds(off, 16)], key, mask=m)
            plsc.store_compressed(sflat.at[pl.ds(off, 16)], sidx, mask=m)
            plsc.store_compressed(tflat.at[pl.ds(off, 16)], tidx, mask=m)
            return off + jnp.sum(m.astype(jnp.int32))

        return lax.fori_loop(0, CHUNK // 16, vec, off)

    cnt = lax.fori_loop(0, NCH, do_chunk, 0)

    dumpv = jnp.full((16,), DUMPSLOT, jnp.int32)
    zv = jnp.zeros((16,), jnp.int32)
    for j in range(8):
        keyflat[pl.ds(cnt + j * 16, 16)] = dumpv
        sflat[pl.ds(cnt + j * 16, 16)] = zv
        tflat[pl.ds(cnt + j * 16, 16)] = zv

    cntbuf[...] = jnp.full((16,), cnt, jnp.int32)
    pltpu.sync_copy(cntbuf, cnts_hbm.at[w])
    pltpu.sync_copy(keyflat, keys_hbm.at[w])
    pltpu.sync_copy(sflat, srcs_hbm.at[w])
    pltpu.sync_copy(tflat, tgts_hbm.at[w])

    nrows = (cnt + 127) // 128

    def scat_start(j, _):
        pltpu.make_async_copy(
            ids2d.at[j], table_hbm.at[keyflat.at[pl.ds(j * 128, 128)]],
            sem).start()
        return 0

    def scat_wait(j, _):
        pltpu.make_async_copy(
            ids2d.at[j], table_hbm.at[keyflat.at[pl.ds(j * 128, 128)]],
            sem).wait()
        return 0

    lax.fori_loop(0, nrows, scat_start, 0)
    lax.fori_loop(0, nrows, scat_wait, 0)


def _sqrt_f32(a):
    bits = plsc.bitcast(a, jnp.int32)
    i = jnp.int32(0x5F3759DF) - (bits >> 1)
    y = plsc.bitcast(i, jnp.float32)
    h = 0.5 * a
    y = y * (1.5 - h * y * y)
    y = y * (1.5 - h * y * y)
    y = y * (1.5 - h * y * y)
    return jnp.where(a > 0.0, a * y, 0.0)


def _bf16r(v):
    u = plsc.bitcast(v, jnp.int32)
    r = (u + 0x7FFF + ((u >> 16) & 1)) & jnp.int32(-65536)
    return plsc.bitcast(r, jnp.float32)


def _compute_body(table_hbm, keys_hbm, srcs_hbm, tgts_hbm, cnts_hbm,
                  x_hbm, pid_hbm, batch_hbm, out_hbm,
                  keyv, srcv2, tgtv2, pidv, batchv, idsall,
                  xs0, xt0, xs1, xt1, accA, accC, accR,
                  outbuf, cntbuf, semA, semB, sem2):
    c = lax.axis_index("c")
    s_ = lax.axis_index("s")
    w = c * 16 + s_
    pltpu.sync_copy(cnts_hbm.at[w], cntbuf)
    cnt = cntbuf[...][0]
    pltpu.sync_copy(keys_hbm.at[w], keyv)
    pltpu.sync_copy(srcs_hbm.at[w], srcv2)
    pltpu.sync_copy(tgts_hbm.at[w], tgtv2)
    pltpu.sync_copy(pid_hbm, pidv)
    pltpu.sync_copy(batch_hbm, batchv)

    nrows = (cnt + 127) // 128
    lane = lax.iota(jnp.int32, 16)
    accA[...] = jnp.zeros((16,), jnp.float32)
    accC[...] = jnp.zeros((16,), jnp.float32)
    accR[...] = jnp.zeros((16,), jnp.float32)

    def x_start(j, xs, xt, sem):
        pltpu.make_async_copy(
            x_hbm.at[srcv2.at[pl.ds(j * 128, 128)]], xs, sem).start()
        pltpu.make_async_copy(
            x_hbm.at[tgtv2.at[pl.ds(j * 128, 128)]], xt, sem).start()

    def x_wait(j, xs, xt, sem):
        pltpu.make_async_copy(
            x_hbm.at[srcv2.at[pl.ds(j * 128, 128)]], xs, sem).wait()
        pltpu.make_async_copy(
            x_hbm.at[tgtv2.at[pl.ds(j * 128, 128)]], xt, sem).wait()

    def g_start(j, _):
        pltpu.make_async_copy(
            table_hbm.at[keyv.at[pl.ds(j * 128, 128)]],
            idsall.at[pl.ds(j * 128, 128)], sem2).start()
        return 0

    def g_wait(j, _):
        pltpu.make_async_copy(
            table_hbm.at[keyv.at[pl.ds(j * 128, 128)]],
            idsall.at[pl.ds(j * 128, 128)], sem2).wait()
        return 0

    lax.fori_loop(0, nrows, g_start, 0)

    @pl.when(nrows > 0)
    def _prime():
        x_start(0, xs0, xt0, semA)

    lax.fori_loop(0, nrows, g_wait, 0)

    def do_row(j, xs, xt):
        base = j * 128
        idbase = w * KCAP2 + base

        def grp(i, _):
            eoff = i * 16
            sv = srcv2[pl.ds(base + eoff, 16)]
            tv = tgtv2[pl.ds(base + eoff, 16)]
            ids = idsall[pl.ds(base + eoff, 16)]
            myid = idbase + eoff + lane
            valid_e = (base + eoff + lane) < cnt
            winner = valid_e & (ids == myid)
            ev = eoff + lane
            dotb = jnp.zeros((16,), jnp.float32)
            dotf = jnp.zeros((16,), jnp.float32)
            sqs = jnp.zeros((16,), jnp.float32)
            sqt = jnp.zeros((16,), jnp.float32)
            for dd in range(16):
                dc = jnp.full((16,), dd, jnp.int32)
                xsd = plsc.load_gather(xs, [ev, dc])
                xtd = plsc.load_gather(xt, [ev, dc])
                dotb = dotb + _bf16r(xsd) * _bf16r(xtd)
                dotf = dotf + xsd * xtd
                sqs = sqs + xsd * xsd
                sqt = sqt + xtd * xtd
            d2m = jnp.maximum(sqs + sqt - 2.0 * dotb, 0.0)
            d2f = jnp.maximum(sqs + sqt - 2.0 * dotf, 0.0)
            dist = _sqrt_f32(d2f)
            bs = plsc.load_gather(batchv, [sv])
            bt = plsc.load_gather(batchv, [tv])
            ps = plsc.load_gather(pidv, [sv])
            pt_ = plsc.load_gather(pidv, [tv])
            inr = (bs == bt) & (d2m <= R2) & (sv != tv)
            rawt = (ps == pt_) & (ps > 0)
            contrib = winner & jnp.logical_not(inr)
            ac = contrib & rawt
            rc = contrib & jnp.logical_not(rawt)
            accA[...] = accA[...] + jnp.where(ac, dist, 0.0)
            accC[...] = accC[...] + jnp.where(ac, 1.0, 0.0)
            accR[...] = accR[...] + jnp.where(
                rc, jnp.maximum(1.0 - dist, 0.0), 0.0)
            return 0

        lax.fori_loop(0, 8, grp, 0)

    def pair(k, _):
        j0 = k * 2
        j1 = j0 + 1

        @pl.when(j0 < nrows)
        def _even():
            x_wait(j0, xs0, xt0, semA)

            @pl.when(j1 < nrows)
            def _pf1():
                x_start(j1, xs1, xt1, semB)

            do_row(j0, xs0, xt0)

        @pl.when(j1 < nrows)
        def _odd():
            x_wait(j1, xs1, xt1, semB)

            @pl.when(j1 + 1 < nrows)
            def _pf2():
                x_start(j1 + 1, xs0, xt0, semA)

            do_row(j1, xs1, xt1)

        return 0

    lax.fori_loop(0, (nrows + 1) // 2, pair, 0)

    attr_s = jnp.sum(accA[...], axis=0)
    cnt_s = jnp.sum(accC[...], axis=0)
    rep_s = jnp.sum(accR[...], axis=0)
    li = lax.iota(jnp.int32, 16)
    outv = jnp.where(li == 0, attr_s,
                     jnp.where(li == 1, cnt_s,
                               jnp.where(li == 2, rep_s, 0.0)))
    outbuf[...] = outv
    pltpu.sync_copy(outbuf, out_hbm.at[w])


def _true_edge_partials(srcp, tgtp, ptp, x128, pid_p, batch_p):
    scatter = functools.partial(
        pl.kernel,
        out_type=(
            jax.ShapeDtypeStruct((NPAD * NPAD,), jnp.int32),
            jax.ShapeDtypeStruct((NW, KCAP2), jnp.int32),
            jax.ShapeDtypeStruct((NW, KCAP2), jnp.int32),
            jax.ShapeDtypeStruct((NW, KCAP2), jnp.int32),
            jax.ShapeDtypeStruct((NW, 16), jnp.int32),
        ),
        mesh=_mesh,
        scratch_types=[
            pltpu.VMEM((NPAD,), jnp.float32),
            pltpu.VMEM((CHUNK,), jnp.int32),
            pltpu.VMEM((CHUNK,), jnp.int32),
            pltpu.VMEM((KCAP2,), jnp.int32),
            pltpu.VMEM((KCAP2,), jnp.int32),
            pltpu.VMEM((KCAP2,), jnp.int32),
            pltpu.VMEM((KCAP2 // 128, 128), jnp.int32),
            pltpu.VMEM((16,), jnp.int32),
            pltpu.SemaphoreType.DMA,
        ],
        compiler_params=_sc_params,
    )(_scatter_body)
    table, keys, srcs, tgts, cnts = scatter(srcp, tgtp, ptp)

    compute = functools.partial(
        pl.kernel,
        out_type=jax.ShapeDtypeStruct((NW, 16), jnp.float32),
        mesh=_mesh,
        scratch_types=[
            pltpu.VMEM((KCAP2,), jnp.int32),
            pltpu.VMEM((KCAP2,), jnp.int32),
            pltpu.VMEM((KCAP2,), jnp.int32),
            pltpu.VMEM((NPAD,), jnp.int32),
            pltpu.VMEM((NPAD,), jnp.int32),
            pltpu.VMEM((KCAP2,), jnp.int32),
            pltpu.VMEM((128, 128), jnp.float32),
            pltpu.VMEM((128, 128), jnp.float32),
            pltpu.VMEM((128, 128), jnp.float32),
            pltpu.VMEM((128, 128), jnp.float32),
            pltpu.VMEM((16,), jnp.float32),
            pltpu.VMEM((16,), jnp.float32),
            pltpu.VMEM((16,), jnp.float32),
            pltpu.VMEM((16,), jnp.float32),
            pltpu.VMEM((16,), jnp.int32),
            pltpu.SemaphoreType.DMA,
            pltpu.SemaphoreType.DMA,
            pltpu.SemaphoreType.DMA,
        ],
        compiler_params=_sc_params,
    )(_compute_body)
    return compute(table, keys, srcs, tgts, cnts, x128, pid_p, batch_p)


def _dense_body(xt_ref, xs_ref, bc_ref, br_ref, pc_ref, pr_ref, ptr_ref,
                attr_ref, cnt_ref, rep_ref, acc_ref):
    rt = pl.program_id(0)
    cs = pl.program_id(1)
    n_t = pl.num_programs(0)
    n_s = pl.num_programs(1)

    @pl.when((rt == 0) & (cs == 0))
    def _init():
        acc_ref[0] = 0.0
        acc_ref[1] = 0.0
        acc_ref[2] = 0.0

    bc = bc_ref[...]
    br = br_ref[...]
    active = (jnp.min(bc) <= jnp.max(br)) & (jnp.min(br) <= jnp.max(bc))

    @pl.when(active)
    def _compute():
        xt = xt_ref[...]
        xs = xs_ref[...]
        sqt = jnp.sum(xt * xt, axis=1, keepdims=True)
        sqs = jnp.sum(xs * xs, axis=1).reshape(1, BS)
        dotb = lax.dot_general(xt.astype(jnp.bfloat16), xs.astype(jnp.bfloat16),
                               (((1,), (1,)), ((), ())),
                               preferred_element_type=jnp.float32)
        d2m = jnp.maximum(sqt + sqs - 2.0 * dotb, 0.0)
        dot = lax.dot_general(xt, xs, (((1,), (1,)), ((), ())),
                              preferred_element_type=jnp.float32,
                              precision=lax.Precision.HIGHEST)
        d2 = jnp.maximum(sqt + sqs - 2.0 * dot, 0.0)
        dist = jnp.sqrt(d2)
        tglob = rt * BT + lax.broadcasted_iota(jnp.int32, (BT, BS), 0)
        sglob = cs * BS + lax.broadcasted_iota(jnp.int32, (BT, BS), 1)
        inr = (bc == br) & (d2m <= R2) & (tglob != sglob)
        rawt = (pc_ref[...] == pr_ref[...]) & (pc_ref[...] > 0)
        ptm = ptr_ref[...] > PT_THLD
        ac = inr & rawt & ptm
        rc = inr & jnp.logical_not(rawt)
        acc_ref[0] += jnp.sum(jnp.where(ac, dist, 0.0))
        acc_ref[1] += jnp.sum(jnp.where(ac, 1.0, 0.0))
        acc_ref[2] += jnp.sum(jnp.where(rc, jnp.maximum(1.0 - dist, 0.0), 0.0))

    @pl.when((rt == n_t - 1) & (cs == n_s - 1))
    def _fin():
        attr_ref[...] = jnp.full((1, 1), acc_ref[0], jnp.float32)
        cnt_ref[...] = jnp.full((1, 1), acc_ref[1], jnp.float32)
        rep_ref[...] = jnp.full((1, 1), acc_ref[2], jnp.float32)


def _radius_sums(xp, batch_p, pid_p, pt_p):
    grid = (NPAD // BT, NPAD // BS)
    bc = batch_p.reshape(NPAD, 1)
    br = batch_p.reshape(1, NPAD)
    pc = pid_p.reshape(NPAD, 1)
    pr = pid_p.reshape(1, NPAD)
    ptr = pt_p.reshape(1, NPAD)
    return pl.pallas_call(
        _dense_body,
        grid=grid,
        in_specs=[
            pl.BlockSpec((BT, 16), lambda i, j: (i, 0)),
            pl.BlockSpec((BS, 16), lambda i, j: (j, 0)),
            pl.BlockSpec((BT, 1), lambda i, j: (i, 0)),
            pl.BlockSpec((1, BS), lambda i, j: (0, j)),
            pl.BlockSpec((BT, 1), lambda i, j: (i, 0)),
            pl.BlockSpec((1, BS), lambda i, j: (0, j)),
            pl.BlockSpec((1, BS), lambda i, j: (0, j)),
        ],
        out_specs=[
            pl.BlockSpec((1, 1), lambda i, j: (0, 0)),
            pl.BlockSpec((1, 1), lambda i, j: (0, 0)),
            pl.BlockSpec((1, 1), lambda i, j: (0, 0)),
        ],
        out_shape=[
            jax.ShapeDtypeStruct((1, 1), jnp.float32),
            jax.ShapeDtypeStruct((1, 1), jnp.float32),
            jax.ShapeDtypeStruct((1, 1), jnp.float32),
        ],
        scratch_shapes=[pltpu.SMEM((4,), jnp.float32)],
        compiler_params=pltpu.CompilerParams(
            dimension_semantics=("arbitrary", "arbitrary")),
    )(xp, xp, bc, br, pc, pr, ptr)


def kernel(x, particle_id, batch, true_edge_index, pt):
    npad = NPAD - N
    pad_x = (1.0e4 + 100.0 * jnp.arange(npad, dtype=jnp.float32))[:, None]
    pad_x = jnp.broadcast_to(pad_x, (npad, x.shape[1]))
    xp = jnp.concatenate([x, pad_x], axis=0)
    batch_p = jnp.concatenate(
        [batch.astype(jnp.int32), jnp.full((npad,), 8, jnp.int32)])
    pid_p = jnp.concatenate(
        [particle_id.astype(jnp.int32), jnp.zeros((npad,), jnp.int32)])
    pt_p = jnp.concatenate([pt, jnp.zeros((npad,), jnp.float32)])
    epad = EPAD - E
    srcp = jnp.concatenate(
        [true_edge_index[0].astype(jnp.int32),
         jnp.full((epad,), NPAD - 1, jnp.int32)])
    tgtp = jnp.concatenate(
        [true_edge_index[1].astype(jnp.int32),
         jnp.full((epad,), NPAD - 2, jnp.int32)])
    x128 = jnp.pad(xp, ((0, 0), (0, 112)))

    partials = _true_edge_partials(srcp, tgtp, pt_p, x128, pid_p, batch_p)
    a_tc, c_tc, r_tc = _radius_sums(xp, batch_p, pid_p, pt_p)

    a_sc = jnp.sum(partials[:, 0])
    c_sc = jnp.sum(partials[:, 1])
    r_sc = jnp.sum(partials[:, 2])
    norm = c_tc[0, 0] + c_sc + 1e-8
    attr = (a_tc[0, 0] + a_sc) / norm
    rep = (r_tc[0, 0] + r_sc) / norm
    return attr, rep


# speedup vs baseline: 1.0011x; 1.0011x over previous
import functools

import jax
import jax.numpy as jnp
from jax import lax
from jax.experimental import pallas as pl
from jax.experimental.pallas import tpu as pltpu
from jax.experimental.pallas import tpu_sc as plsc

N = 10000
NPAD = 10240
E = 320000
EPAD = 327680
NW = 32
EPW = EPAD // NW
CHUNK = 2048
NCH = EPW // CHUNK
KCAP2 = 10496
DUMPSLOT = NPAD * NPAD - 1
R2 = 1.0
PT_THLD = 0.9
BT = 512
BS = 512

_mesh = plsc.VectorSubcoreMesh(core_axis_name="c", subcore_axis_name="s")
_sc_params = pltpu.CompilerParams(needs_layout_passes=False)


def _scatter_body(src_hbm, tgt_hbm, pt_hbm,
                  table_hbm, keys_hbm, cnts_hbm,
                  ptv, srcv, tgtv, keyflat, ids2d, cntbuf, sem):
    c = lax.axis_index("c")
    s_ = lax.axis_index("s")
    w = c * 16 + s_
    pltpu.sync_copy(pt_hbm, ptv)
    ebase = w * EPW

    def fill_ids(j, _):
        for k in range(8):
            ids2d[j, pl.ds(k * 16, 16)] = (
                lax.iota(jnp.int32, 16) + (w * KCAP2 + j * 128 + k * 16))
        return 0

    lax.fori_loop(0, KCAP2 // 128, fill_ids, 0)

    def do_chunk(ch, off):
        pltpu.sync_copy(src_hbm.at[pl.ds(ebase + ch * CHUNK, CHUNK)], srcv)
        pltpu.sync_copy(tgt_hbm.at[pl.ds(ebase + ch * CHUNK, CHUNK)], tgtv)

        def vec(i, off):
            sidx = srcv[pl.ds(i * 16, 16)]
            tidx = tgtv[pl.ds(i * 16, 16)]
            ptg = plsc.load_gather(ptv, [sidx])
            m = ptg > PT_THLD
            key = tidx * NPAD + sidx
            plsc.store_compressed(keyflat.at[pl.ds(off, 16)], key, mask=m)
            return off + jnp.sum(m.astype(jnp.int32))

        return lax.fori_loop(0, CHUNK // 16, vec, off)

    cnt = lax.fori_loop(0, NCH, do_chunk, 0)

    dumpv = jnp.full((16,), DUMPSLOT, jnp.int32)
    for j in range(8):
        keyflat[pl.ds(cnt + j * 16, 16)] = dumpv

    cntbuf[...] = jnp.full((16,), cnt, jnp.int32)
    pltpu.sync_copy(cntbuf, cnts_hbm.at[w])
    pltpu.sync_copy(keyflat, keys_hbm.at[w])

    nrows = (cnt + 127) // 128

    def scat_start(j, _):
        pltpu.make_async_copy(
            ids2d.at[j], table_hbm.at[keyflat.at[pl.ds(j * 128, 128)]],
            sem).start()
        return 0

    def scat_wait(j, _):
        pltpu.make_async_copy(
            ids2d.at[j], table_hbm.at[keyflat.at[pl.ds(j * 128, 128)]],
            sem).wait()
        return 0

    lax.fori_loop(0, nrows, scat_start, 0)
    lax.fori_loop(0, nrows, scat_wait, 0)


def _sqrt_f32(a):
    bits = plsc.bitcast(a, jnp.int32)
    i = jnp.int32(0x5F3759DF) - (bits >> 1)
    y = plsc.bitcast(i, jnp.float32)
    h = 0.5 * a
    y = y * (1.5 - h * y * y)
    y = y * (1.5 - h * y * y)
    y = y * (1.5 - h * y * y)
    return jnp.where(a > 0.0, a * y, 0.0)


def _bf16r(v):
    u = plsc.bitcast(v, jnp.int32)
    r = (u + 0x7FFF + ((u >> 16) & 1)) & jnp.int32(-65536)
    return plsc.bitcast(r, jnp.float32)


def _compute_body(table_hbm, keys_hbm, cnts_hbm,
                  x_hbm, pid_hbm, batch_hbm, out_hbm,
                  keyv, srcv2, tgtv2, pidv, batchv, idsall,
                  xs0, xt0, xs1, xt1, accA, accC, accR,
                  outbuf, cntbuf, semA, semB, sem2):
    c = lax.axis_index("c")
    s_ = lax.axis_index("s")
    w = c * 16 + s_
    pltpu.sync_copy(cnts_hbm.at[w], cntbuf)
    cnt = cntbuf[...][0]
    pltpu.sync_copy(keys_hbm.at[w], keyv)
    pltpu.sync_copy(pid_hbm, pidv)
    pltpu.sync_copy(batch_hbm, batchv)

    nrows = (cnt + 127) // 128
    lane = lax.iota(jnp.int32, 16)

    def unpack(i, _):
        kv = keyv[pl.ds(i * 16, 16)]
        tvv = kv // NPAD
        svv = kv - tvv * NPAD
        srcv2[pl.ds(i * 16, 16)] = svv
        tgtv2[pl.ds(i * 16, 16)] = tvv
        return 0

    lax.fori_loop(0, nrows * 8, unpack, 0)
    accA[...] = jnp.zeros((16,), jnp.float32)
    accC[...] = jnp.zeros((16,), jnp.float32)
    accR[...] = jnp.zeros((16,), jnp.float32)

    def x_start(j, xs, xt, sem):
        pltpu.make_async_copy(
            x_hbm.at[srcv2.at[pl.ds(j * 128, 128)]], xs, sem).start()
        pltpu.make_async_copy(
            x_hbm.at[tgtv2.at[pl.ds(j * 128, 128)]], xt, sem).start()

    def x_wait(j, xs, xt, sem):
        pltpu.make_async_copy(
            x_hbm.at[srcv2.at[pl.ds(j * 128, 128)]], xs, sem).wait()
        pltpu.make_async_copy(
            x_hbm.at[tgtv2.at[pl.ds(j * 128, 128)]], xt, sem).wait()

    def g_start(j, _):
        pltpu.make_async_copy(
            table_hbm.at[keyv.at[pl.ds(j * 128, 128)]],
            idsall.at[pl.ds(j * 128, 128)], sem2).start()
        return 0

    def g_wait(j, _):
        pltpu.make_async_copy(
            table_hbm.at[keyv.at[pl.ds(j * 128, 128)]],
            idsall.at[pl.ds(j * 128, 128)], sem2).wait()
        return 0

    lax.fori_loop(0, nrows, g_start, 0)

    @pl.when(nrows > 0)
    def _prime():
        x_start(0, xs0, xt0, semA)

    lax.fori_loop(0, nrows, g_wait, 0)

    def do_row(j, xs, xt):
        base = j * 128
        idbase = w * KCAP2 + base

        def grp(i, _):
            eoff = i * 16
            sv = srcv2[pl.ds(base + eoff, 16)]
            tv = tgtv2[pl.ds(base + eoff, 16)]
            ids = idsall[pl.ds(base + eoff, 16)]
            myid = idbase + eoff + lane
            valid_e = (base + eoff + lane) < cnt
            winner = valid_e & (ids == myid)
            ev = eoff + lane
            dotb = jnp.zeros((16,), jnp.float32)
            dotf = jnp.zeros((16,), jnp.float32)
            sqs = jnp.zeros((16,), jnp.float32)
            sqt = jnp.zeros((16,), jnp.float32)
            for dd in range(16):
                dc = jnp.full((16,), dd, jnp.int32)
                xsd = plsc.load_gather(xs, [ev, dc])
                xtd = plsc.load_gather(xt, [ev, dc])
                dotb = dotb + _bf16r(xsd) * _bf16r(xtd)
                dotf = dotf + xsd * xtd
                sqs = sqs + xsd * xsd
                sqt = sqt + xtd * xtd
            d2m = jnp.maximum(sqs + sqt - 2.0 * dotb, 0.0)
            d2f = jnp.maximum(sqs + sqt - 2.0 * dotf, 0.0)
            dist = _sqrt_f32(d2f)
            bs = plsc.load_gather(batchv, [sv])
            bt = plsc.load_gather(batchv, [tv])
            ps = plsc.load_gather(pidv, [sv])
            pt_ = plsc.load_gather(pidv, [tv])
            inr = (bs == bt) & (d2m <= R2) & (sv != tv)
            rawt = (ps == pt_) & (ps > 0)
            contrib = winner & jnp.logical_not(inr)
            ac = contrib & rawt
            rc = contrib & jnp.logical_not(rawt)
            accA[...] = accA[...] + jnp.where(ac, dist, 0.0)
            accC[...] = accC[...] + jnp.where(ac, 1.0, 0.0)
            accR[...] = accR[...] + jnp.where(
                rc, jnp.maximum(1.0 - dist, 0.0), 0.0)
            return 0

        lax.fori_loop(0, 8, grp, 0)

    def pair(k, _):
        j0 = k * 2
        j1 = j0 + 1

        @pl.when(j0 < nrows)
        def _even():
            x_wait(j0, xs0, xt0, semA)

            @pl.when(j1 < nrows)
            def _pf1():
                x_start(j1, xs1, xt1, semB)

            do_row(j0, xs0, xt0)

        @pl.when(j1 < nrows)
        def _odd():
            x_wait(j1, xs1, xt1, semB)

            @pl.when(j1 + 1 < nrows)
            def _pf2():
                x_start(j1 + 1, xs0, xt0, semA)

            do_row(j1, xs1, xt1)

        return 0

    lax.fori_loop(0, (nrows + 1) // 2, pair, 0)

    attr_s = jnp.sum(accA[...], axis=0)
    cnt_s = jnp.sum(accC[...], axis=0)
    rep_s = jnp.sum(accR[...], axis=0)
    li = lax.iota(jnp.int32, 16)
    outv = jnp.where(li == 0, attr_s,
                     jnp.where(li == 1, cnt_s,
                               jnp.where(li == 2, rep_s, 0.0)))
    outbuf[...] = outv
    pltpu.sync_copy(outbuf, out_hbm.at[w])


def _true_edge_partials(srcp, tgtp, ptp, x128, pid_p, batch_p):
    scatter = functools.partial(
        pl.kernel,
        out_type=(
            jax.ShapeDtypeStruct((NPAD * NPAD,), jnp.int32),
            jax.ShapeDtypeStruct((NW, KCAP2), jnp.int32),
            jax.ShapeDtypeStruct((NW, 16), jnp.int32),
        ),
        mesh=_mesh,
        scratch_types=[
            pltpu.VMEM((NPAD,), jnp.float32),
            pltpu.VMEM((CHUNK,), jnp.int32),
            pltpu.VMEM((CHUNK,), jnp.int32),
            pltpu.VMEM((KCAP2,), jnp.int32),
            pltpu.VMEM((KCAP2 // 128, 128), jnp.int32),
            pltpu.VMEM((16,), jnp.int32),
            pltpu.SemaphoreType.DMA,
        ],
        compiler_params=_sc_params,
    )(_scatter_body)
    table, keys, cnts = scatter(srcp, tgtp, ptp)

    compute = functools.partial(
        pl.kernel,
        out_type=jax.ShapeDtypeStruct((NW, 16), jnp.float32),
        mesh=_mesh,
        scratch_types=[
            pltpu.VMEM((KCAP2,), jnp.int32),
            pltpu.VMEM((KCAP2,), jnp.int32),
            pltpu.VMEM((KCAP2,), jnp.int32),
            pltpu.VMEM((NPAD,), jnp.int32),
            pltpu.VMEM((NPAD,), jnp.int32),
            pltpu.VMEM((KCAP2,), jnp.int32),
            pltpu.VMEM((128, 128), jnp.float32),
            pltpu.VMEM((128, 128), jnp.float32),
            pltpu.VMEM((128, 128), jnp.float32),
            pltpu.VMEM((128, 128), jnp.float32),
            pltpu.VMEM((16,), jnp.float32),
            pltpu.VMEM((16,), jnp.float32),
            pltpu.VMEM((16,), jnp.float32),
            pltpu.VMEM((16,), jnp.float32),
            pltpu.VMEM((16,), jnp.int32),
            pltpu.SemaphoreType.DMA,
            pltpu.SemaphoreType.DMA,
            pltpu.SemaphoreType.DMA,
        ],
        compiler_params=_sc_params,
    )(_compute_body)
    return compute(table, keys, cnts, x128, pid_p, batch_p)


def _dense_body(xt_ref, xs_ref, bc_ref, br_ref, pc_ref, pr_ref, ptr_ref,
                attr_ref, cnt_ref, rep_ref, acc_ref):
    rt = pl.program_id(0)
    cs = pl.program_id(1)
    n_t = pl.num_programs(0)
    n_s = pl.num_programs(1)

    @pl.when((rt == 0) & (cs == 0))
    def _init():
        acc_ref[0] = 0.0
        acc_ref[1] = 0.0
        acc_ref[2] = 0.0

    bc = bc_ref[...]
    br = br_ref[...]
    active = (jnp.min(bc) <= jnp.max(br)) & (jnp.min(br) <= jnp.max(bc))

    @pl.when(active)
    def _compute():
        xt = xt_ref[...]
        xs = xs_ref[...]
        sqt = jnp.sum(xt * xt, axis=1, keepdims=True)
        sqs = jnp.sum(xs * xs, axis=1).reshape(1, BS)
        dotb = lax.dot_general(xt.astype(jnp.bfloat16), xs.astype(jnp.bfloat16),
                               (((1,), (1,)), ((), ())),
                               preferred_element_type=jnp.float32)
        d2m = jnp.maximum(sqt + sqs - 2.0 * dotb, 0.0)
        dot = lax.dot_general(xt, xs, (((1,), (1,)), ((), ())),
                              preferred_element_type=jnp.float32,
                              precision=lax.Precision.HIGHEST)
        d2 = jnp.maximum(sqt + sqs - 2.0 * dot, 0.0)
        dist = jnp.sqrt(d2)
        tglob = rt * BT + lax.broadcasted_iota(jnp.int32, (BT, BS), 0)
        sglob = cs * BS + lax.broadcasted_iota(jnp.int32, (BT, BS), 1)
        inr = (bc == br) & (d2m <= R2) & (tglob != sglob)
        rawt = (pc_ref[...] == pr_ref[...]) & (pc_ref[...] > 0)
        ptm = ptr_ref[...] > PT_THLD
        ac = inr & rawt & ptm
        rc = inr & jnp.logical_not(rawt)
        acc_ref[0] += jnp.sum(jnp.where(ac, dist, 0.0))
        acc_ref[1] += jnp.sum(jnp.where(ac, 1.0, 0.0))
        acc_ref[2] += jnp.sum(jnp.where(rc, jnp.maximum(1.0 - dist, 0.0), 0.0))

    @pl.when((rt == n_t - 1) & (cs == n_s - 1))
    def _fin():
        attr_ref[...] = jnp.full((1, 1), acc_ref[0], jnp.float32)
        cnt_ref[...] = jnp.full((1, 1), acc_ref[1], jnp.float32)
        rep_ref[...] = jnp.full((1, 1), acc_ref[2], jnp.float32)


def _radius_sums(xp, batch_p, pid_p, pt_p):
    grid = (NPAD // BT, NPAD // BS)
    bc = batch_p.reshape(NPAD, 1)
    br = batch_p.reshape(1, NPAD)
    pc = pid_p.reshape(NPAD, 1)
    pr = pid_p.reshape(1, NPAD)
    ptr = pt_p.reshape(1, NPAD)
    return pl.pallas_call(
        _dense_body,
        grid=grid,
        in_specs=[
            pl.BlockSpec((BT, 16), lambda i, j: (i, 0)),
            pl.BlockSpec((BS, 16), lambda i, j: (j, 0)),
            pl.BlockSpec((BT, 1), lambda i, j: (i, 0)),
            pl.BlockSpec((1, BS), lambda i, j: (0, j)),
            pl.BlockSpec((BT, 1), lambda i, j: (i, 0)),
            pl.BlockSpec((1, BS), lambda i, j: (0, j)),
            pl.BlockSpec((1, BS), lambda i, j: (0, j)),
        ],
        out_specs=[
            pl.BlockSpec((1, 1), lambda i, j: (0, 0)),
            pl.BlockSpec((1, 1), lambda i, j: (0, 0)),
            pl.BlockSpec((1, 1), lambda i, j: (0, 0)),
        ],
        out_shape=[
            jax.ShapeDtypeStruct((1, 1), jnp.float32),
            jax.ShapeDtypeStruct((1, 1), jnp.float32),
            jax.ShapeDtypeStruct((1, 1), jnp.float32),
        ],
        scratch_shapes=[pltpu.SMEM((4,), jnp.float32)],
        compiler_params=pltpu.CompilerParams(
            dimension_semantics=("arbitrary", "arbitrary")),
    )(xp, xp, bc, br, pc, pr, ptr)


def kernel(x, particle_id, batch, true_edge_index, pt):
    npad = NPAD - N
    pad_x = (1.0e4 + 100.0 * jnp.arange(npad, dtype=jnp.float32))[:, None]
    pad_x = jnp.broadcast_to(pad_x, (npad, x.shape[1]))
    xp = jnp.concatenate([x, pad_x], axis=0)
    batch_p = jnp.concatenate(
        [batch.astype(jnp.int32), jnp.full((npad,), 8, jnp.int32)])
    pid_p = jnp.concatenate(
        [particle_id.astype(jnp.int32), jnp.zeros((npad,), jnp.int32)])
    pt_p = jnp.concatenate([pt, jnp.zeros((npad,), jnp.float32)])
    epad = EPAD - E
    srcp = jnp.concatenate(
        [true_edge_index[0].astype(jnp.int32),
         jnp.full((epad,), NPAD - 1, jnp.int32)])
    tgtp = jnp.concatenate(
        [true_edge_index[1].astype(jnp.int32),
         jnp.full((epad,), NPAD - 2, jnp.int32)])
    x128 = jnp.pad(xp, ((0, 0), (0, 112)))

    partials = _true_edge_partials(srcp, tgtp, pt_p, x128, pid_p, batch_p)
    a_tc, c_tc, r_tc = _radius_sums(xp, batch_p, pid_p, pt_p)

    a_sc = jnp.sum(partials[:, 0])
    c_sc = jnp.sum(partials[:, 1])
    r_sc = jnp.sum(partials[:, 2])
    norm = c_tc[0, 0] + c_sc + 1e-8
    attr = (a_tc[0, 0] + a_sc) / norm
    rep = (r_tc[0, 0] + r_sc) / norm
    return attr, rep
